# Initial kernel scaffold; baseline (speedup 1.0000x reference)
import jax
import jax.numpy as jnp
from jax.experimental import pallas as pl


def kernel(x, edge_index, W1l, W1r, b1, W2l, W2r, b2):
    raise NotImplementedError("write your pallas kernel here")


# speedup vs baseline: 12.6406x; 12.6406x over previous
import jax
import jax.numpy as jnp
from jax import lax
from jax.experimental import pallas as pl
from jax.experimental.pallas import tpu as pltpu
from jax.experimental.pallas import tpu_sc as plsc

NC = 2
NS = 16
LANES = 128
NBUF = 4


def _proj_body(x_ref, wl_ref, wr_ref, b_ref, ys_ref, self_ref):
    xx = x_ref[...]
    y = jnp.dot(xx, wl_ref[...], preferred_element_type=jnp.float32)
    half = y.shape[1] // 2
    ys_ref[0] = y[:, :half]
    ys_ref[1] = y[:, half:]
    self_ref[...] = (
        jnp.dot(xx, wr_ref[...], preferred_element_type=jnp.float32) + b_ref[...]
    )


def _mid_body(s1_ref, deg_ref, self1_ref, wl_ref, wr_ref, b_ref, y2_ref,
              self2_ref):
    nn = self1_ref.shape[0]
    d = deg_ref[0, :nn, 0:1] + deg_ref[1, :nn, 0:1]
    rec = 1.0 / jnp.maximum(d, 1.0)
    s1 = jnp.concatenate([s1_ref[0, :nn], s1_ref[1, :nn]], axis=1)
    h = jnp.maximum(s1 * rec + self1_ref[...], 0.0)
    y2_ref[...] = jnp.dot(h, wl_ref[...], preferred_element_type=jnp.float32)
    self2_ref[...] = (
        jnp.dot(h, wr_ref[...], preferred_element_type=jnp.float32) + b_ref[...]
    )


def _final_body(s2_ref, deg_ref, self2_ref, out_ref):
    nn = self2_ref.shape[0]
    d = deg_ref[0, :nn, 0:1] + deg_ref[1, :nn, 0:1]
    rec = 1.0 / jnp.maximum(d, 1.0)
    z = (s2_ref[0, :nn] + s2_ref[1, :nn]) * rec + self2_ref[...]
    m = jnp.max(z, axis=1, keepdims=True)
    zs = z - m
    lse = jnp.log(jnp.sum(jnp.exp(zs), axis=1, keepdims=True))
    out_ref[...] = zs - lse


def _npad(n_nodes):
    return ((n_nodes + 1 + 127) // 128) * 128


def _segsum_loop(y_ref, srcbuf, dstbuf, rows, g_sems, s_sems,
                 agg_sh, n_rows, deg=None):
    R = n_rows
    for b in range(NBUF):
        pltpu.async_copy(y_ref.at[srcbuf.at[b]], rows[b], g_sems[b])

    def group(i, carry):
        for b in range(NBUF):
            g = i * NBUF + b
            pltpu.make_async_copy(y_ref.at[srcbuf.at[g]], rows[b],
                                  g_sems[b]).wait()
            pltpu.async_copy(rows[b], agg_sh.at[dstbuf.at[g]], s_sems[b],
                             add=True)
            if deg is not None:
                ones_v, deg_sh, d_sem, cond_fn = deg

                @pl.when(cond_fn(g))
                def _():
                    pltpu.async_copy(ones_v, deg_sh.at[dstbuf.at[g]], d_sem,
                                     add=True)

            b2 = (b + 2) % NBUF

            @pl.when((g >= 2) & (g + 2 < R))
            def _():
                pltpu.make_async_copy(rows[b2], agg_sh.at[dstbuf.at[0]],
                                      s_sems[b2]).wait()
                pltpu.async_copy(y_ref.at[srcbuf.at[g + 2]], rows[b2],
                                 g_sems[b2])
        return carry

    lax.fori_loop(0, R // NBUF, group, 0)
    for b in range(NBUF):
        pltpu.make_async_copy(rows[b], agg_sh.at[dstbuf.at[0]],
                              s_sems[b]).wait()
    if deg is not None:
        ones_v, deg_sh, d_sem, _ = deg

        def dwait(i, carry):
            pltpu.make_async_copy(ones_v, deg_sh.at[dstbuf.at[0]],
                                  d_sem).wait()
            return carry

        lax.fori_loop(0, R // 2, dwait, 0)


def _make_sc_layer1(n_nodes, half, rows_per_tile):
    R = rows_per_tile
    NPAD = _npad(n_nodes)
    ZR = NPAD // NS

    def body(ys_hbm, src_hbm, dst_hbm, zd_hbm, z16_hbm, ones_hbm,
             out_hbm, deg_hbm,
             srcbuf, dstbuf, r0, r1, r2, r3, ones_v,
             agg_sh, deg_sh, g0, g1, g2, g3, s0, s1, s2, s3, dsem):
        c = lax.axis_index("c")
        s = lax.axis_index("s")

        pltpu.sync_copy(zd_hbm, agg_sh.at[pl.ds(s * ZR, ZR)])
        pltpu.sync_copy(z16_hbm, deg_sh.at[pl.ds(s * ZR, ZR)])
        pltpu.sync_copy(ones_hbm, ones_v)
        pltpu.sync_copy(src_hbm.at[pl.ds(s * R, R)], srcbuf)
        pltpu.sync_copy(dst_hbm.at[pl.ds(s * R, R)], dstbuf)
        plsc.subcore_barrier()

        rows = [r0, r1, r2, r3]
        g_sems = [g0, g1, g2, g3]
        s_sems = [s0, s1, s2, s3]
        hR = R // 2

        def cond_fn(g):
            return lax.select(c == 0, g < hR, g >= hR)

        _segsum_loop(ys_hbm.at[c], srcbuf, dstbuf, rows, g_sems, s_sems,
                     agg_sh, R, deg=(ones_v, deg_sh, dsem, cond_fn))

        plsc.subcore_barrier()
        pltpu.sync_copy(agg_sh.at[pl.ds(s * ZR, ZR)],
                        out_hbm.at[c, pl.ds(s * ZR, ZR)])
        pltpu.sync_copy(deg_sh.at[pl.ds(s * ZR, ZR)],
                        deg_hbm.at[c, pl.ds(s * ZR, ZR)])

    out_type = [
        jax.ShapeDtypeStruct((NC, NPAD, half), jnp.float32),
        jax.ShapeDtypeStruct((NC, NPAD, 16), jnp.float32),
    ]
    scratch = [
        pltpu.VMEM((R, LANES), jnp.int32),
        pltpu.VMEM((R, LANES), jnp.int32),
        pltpu.VMEM((LANES, half), jnp.float32),
        pltpu.VMEM((LANES, half), jnp.float32),
        pltpu.VMEM((LANES, half), jnp.float32),
        pltpu.VMEM((LANES, half), jnp.float32),
        pltpu.VMEM((LANES, 16), jnp.float32),
        pltpu.VMEM_SHARED((NPAD, half), jnp.float32),
        pltpu.VMEM_SHARED((NPAD, 16), jnp.float32),
    ] + [pltpu.SemaphoreType.DMA] * 9
    mesh = plsc.VectorSubcoreMesh(core_axis_name="c", subcore_axis_name="s")
    return pl.kernel(
        body, out_type=out_type, mesh=mesh, scratch_types=scratch,
        compiler_params=pltpu.CompilerParams(use_tc_tiling_on_sc=False))


def _make_sc_layer2(n_nodes, d, rows_per_tile):
    R = rows_per_tile
    NPAD = _npad(n_nodes)
    ZR = NPAD // NS

    def body(y_hbm, src_hbm, dst_hbm, zd_hbm,
             out_hbm,
             srcbuf, dstbuf, r0, r1, r2, r3,
             agg_sh, g0, g1, g2, g3, s0, s1, s2, s3):
        c = lax.axis_index("c")
        s = lax.axis_index("s")
        tid = c * NS + s

        pltpu.sync_copy(zd_hbm, agg_sh.at[pl.ds(s * ZR, ZR)])
        pltpu.sync_copy(src_hbm.at[pl.ds(tid * R, R)], srcbuf)
        pltpu.sync_copy(dst_hbm.at[pl.ds(tid * R, R)], dstbuf)
        plsc.subcore_barrier()

        _segsum_loop(y_hbm, srcbuf, dstbuf, [r0, r1, r2, r3],
                     [g0, g1, g2, g3], [s0, s1, s2, s3], agg_sh, R)

        plsc.subcore_barrier()
        pltpu.sync_copy(agg_sh.at[pl.ds(s * ZR, ZR)],
                        out_hbm.at[c, pl.ds(s * ZR, ZR)])

    out_type = jax.ShapeDtypeStruct((NC, NPAD, d), jnp.float32)
    scratch = [
        pltpu.VMEM((R, LANES), jnp.int32),
        pltpu.VMEM((R, LANES), jnp.int32),
        pltpu.VMEM((LANES, d), jnp.float32),
        pltpu.VMEM((LANES, d), jnp.float32),
        pltpu.VMEM((LANES, d), jnp.float32),
        pltpu.VMEM((LANES, d), jnp.float32),
        pltpu.VMEM_SHARED((NPAD, d), jnp.float32),
    ] + [pltpu.SemaphoreType.DMA] * 8
    mesh = plsc.VectorSubcoreMesh(core_axis_name="c", subcore_axis_name="s")
    return pl.kernel(
        body, out_type=out_type, mesh=mesh, scratch_types=scratch,
        compiler_params=pltpu.CompilerParams(use_tc_tiling_on_sc=False))


def kernel(x, edge_index, W1l, W1r, b1, W2l, W2r, b2):
    n, d_in = x.shape
    d_hid = W1l.shape[1]
    n_cls = W2l.shape[1]
    e = edge_index.shape[1]

    src = edge_index[0].astype(jnp.int32)
    dst = edge_index[1].astype(jnp.int32)

    unit = NC * NS * LANES * NBUF
    e_pad = -(-e // unit) * unit
    pad = e_pad - e
    npad = _npad(n)
    pad_ar = jnp.arange(pad, dtype=jnp.int32)
    src_p = jnp.concatenate([src, pad_ar % n])
    dst_p = jnp.concatenate([dst, n + pad_ar % (npad - n)])
    src2d = src_p.reshape(-1, LANES)
    dst2d = dst_p.reshape(-1, LANES)
    r1 = e_pad // (NS * LANES)
    r2 = e_pad // (NC * NS * LANES)

    zr = npad // NS
    half = d_hid // 2
    z_half = jnp.zeros((zr, half), jnp.float32)
    z_cls = jnp.zeros((zr, n_cls), jnp.float32)
    z16 = jnp.zeros((zr, 16), jnp.float32)
    ones16 = jnp.ones((LANES, 16), jnp.float32)

    y1s, self1 = pl.pallas_call(
        _proj_body,
        out_shape=[
            jax.ShapeDtypeStruct((NC, n, half), jnp.float32),
            jax.ShapeDtypeStruct((n, d_hid), jnp.float32),
        ],
    )(x, W1l, W1r, b1.reshape(1, -1))

    sc1 = _make_sc_layer1(n, half, r1)
    s1p, degp = sc1(y1s, src2d, dst2d, z_half, z16, ones16)

    y2, self2 = pl.pallas_call(
        _mid_body,
        out_shape=[
            jax.ShapeDtypeStruct((n, n_cls), jnp.float32),
            jax.ShapeDtypeStruct((n, n_cls), jnp.float32),
        ],
    )(s1p, degp, self1, W2l, W2r, b2.reshape(1, -1))

    sc2 = _make_sc_layer2(n, n_cls, r2)
    s2p = sc2(y2, src2d, dst2d, z_cls)

    out = pl.pallas_call(
        _final_body,
        out_shape=jax.ShapeDtypeStruct((n, n_cls), jnp.float32),
    )(s2p, degp, self2)
    return out
